# baseline (device time: 46859 ns/iter reference)
import jax
import jax.numpy as jnp
from jax import lax
from jax.experimental import pallas as pl
from jax.experimental.pallas import tpu as pltpu

N_DEV = 4


def kernel(x, w_mat, scale_x, scale_w):
    m_tot, k_per = x.shape
    _, n = w_mat.shape
    m_per = m_tot // N_DEV

    def body(x_ref, w_ref, sx_ref, sw_ref, out_ref, xg_ref, send_sems, recv_sems):
        my = lax.axis_index("i")

        barrier_sem = pltpu.get_barrier_semaphore()
        for d in range(1, N_DEV):
            pl.semaphore_signal(
                barrier_sem, inc=1,
                device_id=((my + d) % N_DEV,),
                device_id_type=pl.DeviceIdType.MESH,
            )
        pl.semaphore_wait(barrier_sem, N_DEV - 1)

        sends = []
        for d in range(1, N_DEV):
            peer = (my + d) % N_DEV
            rdma = pltpu.make_async_remote_copy(
                src_ref=x_ref.at[pl.ds(peer * m_per, m_per), :],
                dst_ref=xg_ref.at[my],
                send_sem=send_sems.at[d - 1],
                recv_sem=recv_sems.at[d - 1],
                device_id=(peer,),
                device_id_type=pl.DeviceIdType.MESH,
            )
            rdma.start()
            sends.append(rdma)

        def partial(xs, k_slice):
            return lax.dot_general(
                xs, w_ref[pl.ds(k_slice * k_per, k_per), :],
                (((1,), (0,)), ((), ())),
                preferred_element_type=jnp.int32,
            )

        acc = partial(x_ref[pl.ds(my * m_per, m_per), :], my)

        for d in (1, 3, 2):
            src = (my + N_DEV - d) % N_DEV
            recv = pltpu.make_async_remote_copy(
                src_ref=x_ref.at[pl.ds(0, m_per), :],
                dst_ref=xg_ref.at[src],
                send_sem=send_sems.at[d - 1],
                recv_sem=recv_sems.at[d - 1],
                device_id=(my,),
                device_id_type=pl.DeviceIdType.MESH,
            )
            recv.wait_recv()
            acc = acc + partial(xg_ref[src], src)

        s = sx_ref[0] * sw_ref[0]
        y = acc.astype(jnp.float32) * s
        out_ref[:, :] = y * jax.nn.sigmoid(y)

        for rdma in sends:
            rdma.wait_send()

    return pl.pallas_call(
        body,
        out_shape=jax.ShapeDtypeStruct((m_per, n), jnp.float32),
        in_specs=[
            pl.BlockSpec(memory_space=pltpu.VMEM),
            pl.BlockSpec(memory_space=pltpu.VMEM),
            pl.BlockSpec(memory_space=pltpu.SMEM),
            pl.BlockSpec(memory_space=pltpu.SMEM),
        ],
        out_specs=pl.BlockSpec(memory_space=pltpu.VMEM),
        scratch_shapes=[
            pltpu.VMEM((N_DEV, m_per, k_per), jnp.int8),
            pltpu.SemaphoreType.DMA((N_DEV - 1,)),
            pltpu.SemaphoreType.DMA((N_DEV - 1,)),
        ],
        compiler_params=pltpu.CompilerParams(collective_id=0),
    )(x, w_mat, scale_x, scale_w)


# device time: 38223 ns/iter; 1.2259x vs baseline; 1.2259x over previous
import jax
import jax.numpy as jnp
from jax import lax
from jax.experimental import pallas as pl
from jax.experimental.pallas import tpu as pltpu

N_DEV = 4
SUB = 4


def kernel(x, w_mat, scale_x, scale_w):
    m_tot, k_per = x.shape
    _, n = w_mat.shape
    m_per = m_tot // N_DEV
    m_sub = m_per // SUB

    def body(x_ref, w_ref, sx_ref, sw_ref, out_ref, xg_ref, send_sems, recv_sems):
        my = lax.axis_index("i")

        with jax.named_scope("barrier"):
            barrier_sem = pltpu.get_barrier_semaphore()
            for d in range(1, N_DEV):
                pl.semaphore_signal(
                    barrier_sem, inc=1,
                    device_id=((my + d) % N_DEV,),
                    device_id_type=pl.DeviceIdType.MESH,
                )
            pl.semaphore_wait(barrier_sem, N_DEV - 1)

        sends = []
        with jax.named_scope("send_issue"):
            for k in range(SUB):
                for d in (1, 3, 2):
                    peer = (my + d) % N_DEV
                    rdma = pltpu.make_async_remote_copy(
                        src_ref=x_ref.at[
                            pl.ds(peer * m_per + k * m_sub, m_sub), :],
                        dst_ref=xg_ref.at[my, pl.ds(k * m_sub, m_sub), :],
                        send_sem=send_sems.at[d - 1, k],
                        recv_sem=recv_sems.at[d - 1, k],
                        device_id=(peer,),
                        device_id_type=pl.DeviceIdType.MESH,
                    )
                    rdma.start()
                    sends.append(rdma)

        def partial(xs, k_slice):
            return lax.dot_general(
                xs.astype(jnp.bfloat16),
                w_ref[pl.ds(k_slice * k_per, k_per), :].astype(jnp.bfloat16),
                (((1,), (0,)), ((), ())),
                preferred_element_type=jnp.float32,
            )

        acc = []
        with jax.named_scope("gemm_local"):
            for k in range(SUB):
                acc.append(
                    partial(x_ref[pl.ds(my * m_per + k * m_sub, m_sub), :], my))

        s = sx_ref[0] * sw_ref[0]
        for k in range(SUB):
            for d in (1, 3, 2):
                src = (my + N_DEV - d) % N_DEV
                with jax.named_scope(f"wait_recv#r={k}_d={d}"):
                    recv = pltpu.make_async_remote_copy(
                        src_ref=x_ref.at[pl.ds(0, m_sub), :],
                        dst_ref=xg_ref.at[src, pl.ds(k * m_sub, m_sub), :],
                        send_sem=send_sems.at[d - 1, k],
                        recv_sem=recv_sems.at[d - 1, k],
                        device_id=(my,),
                        device_id_type=pl.DeviceIdType.MESH,
                    )
                    recv.wait_recv()
                with jax.named_scope(f"gemm#r={k}_d={d}"):
                    acc[k] = acc[k] + partial(
                        xg_ref[src, pl.ds(k * m_sub, m_sub), :], src)
            with jax.named_scope(f"epilogue#r={k}"):
                y = acc[k] * s
                out_ref[pl.ds(k * m_sub, m_sub), :] = y * jax.nn.sigmoid(y)

        with jax.named_scope("wait_send"):
            for rdma in sends:
                rdma.wait_send()

    return pl.pallas_call(
        body,
        out_shape=jax.ShapeDtypeStruct((m_per, n), jnp.float32),
        in_specs=[
            pl.BlockSpec(memory_space=pltpu.VMEM),
            pl.BlockSpec(memory_space=pltpu.VMEM),
            pl.BlockSpec(memory_space=pltpu.SMEM),
            pl.BlockSpec(memory_space=pltpu.SMEM),
        ],
        out_specs=pl.BlockSpec(memory_space=pltpu.VMEM),
        scratch_shapes=[
            pltpu.VMEM((N_DEV, m_per, k_per), jnp.int8),
            pltpu.SemaphoreType.DMA((N_DEV - 1, SUB)),
            pltpu.SemaphoreType.DMA((N_DEV - 1, SUB)),
        ],
        compiler_params=pltpu.CompilerParams(collective_id=0),
    )(x, w_mat, scale_x, scale_w)
